# Initial kernel scaffold; baseline (speedup 1.0000x reference)
import jax
import jax.numpy as jnp
from jax.experimental import pallas as pl


def kernel(h_states, end_pos, rel_pos, annotated_points, W_sp, b_sp, W1, b1, W2, b2, seq_start_end):
    raise NotImplementedError("write your pallas kernel here")


# speedup vs baseline: 3.8792x; 3.8792x over previous
import functools

import jax
import jax.numpy as jnp
from jax.experimental import pallas as pl

NEIGH_HALF = 1.0


def _pool_kernel(epx_ref, epy_ref, apx_ref, apy_ref, h_ref, W_sp_ref, b_sp_ref,
                 W1_ref, b1_ref, W2_ref, b2_ref, out_ref, *, tp, nc):
    e64 = W1_ref.shape[0] - h_ref.shape[1]
    W1_top = W1_ref[:e64, :]
    A = jnp.dot(W_sp_ref[...], W1_top, preferred_element_type=jnp.float32)
    base = (jnp.dot(h_ref[...], W1_ref[e64:, :],
                    preferred_element_type=jnp.float32)
            + jnp.dot(b_sp_ref[...], W1_top,
                      preferred_element_type=jnp.float32)
            + b1_ref[...])

    rx = apx_ref[...] - epx_ref[...]
    ry = apy_ref[...] - epy_ref[...]
    rx = jnp.where(jnp.abs(rx) > NEIGH_HALF, 0.0, rx)
    ry = jnp.where(jnp.abs(ry) > NEIGH_HALF, 0.0, ry)

    pre1 = (rx[:, :, None] * A[0][None, None, :]
            + ry[:, :, None] * A[1][None, None, :]
            + base[:, None, :])
    x1 = jnp.maximum(pre1, 0.0).reshape(tp * nc, -1)

    y = jnp.dot(x1, W2_ref[...], preferred_element_type=jnp.float32)
    y = jnp.maximum(y + b2_ref[...], 0.0)
    out_ref[...] = jnp.max(y.reshape(tp, nc, -1), axis=1)


def kernel(h_states, end_pos, rel_pos, annotated_points, W_sp, b_sp, W1, b1,
           W2, b2, seq_start_end):
    del rel_pos, seq_start_end
    h = h_states.reshape(-1, h_states.shape[-1])
    B = h.shape[0]
    NC = annotated_points.shape[0]
    BN = W2.shape[1]
    TP = 16
    grid = (B // TP,)

    epx = end_pos[:, 0:1]
    epy = end_pos[:, 1:2]
    apx = annotated_points[:, 0].reshape(1, NC)
    apy = annotated_points[:, 1].reshape(1, NC)

    full = lambda shape: pl.BlockSpec(shape, lambda i: (0, 0))
    out = pl.pallas_call(
        functools.partial(_pool_kernel, tp=TP, nc=NC),
        grid=grid,
        in_specs=[
            pl.BlockSpec((TP, 1), lambda i: (i, 0)),
            pl.BlockSpec((TP, 1), lambda i: (i, 0)),
            full((1, NC)),
            full((1, NC)),
            pl.BlockSpec((TP, h.shape[1]), lambda i: (i, 0)),
            full(W_sp.shape),
            full((1, b_sp.shape[0])),
            full(W1.shape),
            full((1, b1.shape[0])),
            full(W2.shape),
            full((1, b2.shape[0])),
        ],
        out_specs=pl.BlockSpec((TP, BN), lambda i: (i, 0)),
        out_shape=jax.ShapeDtypeStruct((B, BN), jnp.float32),
    )(epx, epy, apx, apy, h, W_sp, b_sp.reshape(1, -1), W1, b1.reshape(1, -1),
      W2, b2.reshape(1, -1))
    return out
